# baseline (speedup 1.0000x reference)
import functools

import jax
import jax.numpy as jnp
from jax import lax
from jax.experimental import pallas as pl
from jax.experimental.pallas import tpu as pltpu
from jax.experimental.pallas import tpu_sc as plsc

N = 10000
E = 320000
NC = 2
NS = 16
NW = NC * NS
CHUNK = 128
NCHUNK = 80
SUB = 64
NSUB = 160
EPW = NCHUNK * CHUNK
EPAD = NW * EPW
NPAD = 10240
ZROWS = NPAD // NS
DIM = 128


def _seg_sum_body(table_hbm, pk_hbm, out_hbm,
                  pk_idx, srcc, dstc, b0, b1, b2, b3, f0, f1, acc,
                  g0, g1, g2, g3, s0, s1):
    core = lax.axis_index("c")
    sub = lax.axis_index("s")
    wid = core * NS + sub
    bufs = (b0, b1, b2, b3)
    gsem = (g0, g1, g2, g3)
    frows = (f0, f1)
    ssem = (s0, s1)

    def zrow(r, _):
        for cc in range(DIM // 16):
            f0[r, pl.ds(cc * 16, 16)] = jnp.zeros((16,), jnp.float32)
        return 0
    lax.fori_loop(0, SUB, zrow, 0)
    for k in range(ZROWS // SUB):
        pltpu.sync_copy(f0, acc.at[pl.ds(sub * ZROWS + k * SUB, SUB)])
    plsc.subcore_barrier()

    pltpu.sync_copy(pk_hbm.at[wid], pk_idx)

    def unpack_idx(j2, off, gslot, dslot):
        for v in range(SUB // 16):
            p = pk_idx[j2, pl.ds(off + v * 16, 16)]
            srcc[gslot, pl.ds(v * 16, 16)] = p & 0x3FFF
            dstc[dslot, pl.ds(v * 16, 16)] = p >> 14

    def fire_gather(slot):
        pltpu.async_copy(table_hbm.at[srcc.at[slot]], bufs[slot], gsem[slot])

    def wait_gather(slot):
        pltpu.make_async_copy(table_hbm.at[srcc.at[slot]], bufs[slot],
                              gsem[slot]).wait()

    def convert(gi, s):
        def crow(r, _):
            for g in range(DIM // 32):
                w = bufs[gi][r, pl.ds(16 * g, 16)]
                a = plsc.bitcast(w << 16, jnp.float32)
                b = plsc.bitcast(w & jnp.int32(-65536), jnp.float32)
                frows[s][r, pl.ds(32 * g, 16)] = a
                frows[s][r, pl.ds(32 * g + 16, 16)] = b
            return 0
        lax.fori_loop(0, SUB, crow, 0)

    def fire_scatter(s, d8):
        pltpu.async_copy(frows[s], acc.at[dstc.at[d8]], ssem[s], add=True)

    def wait_scatter(s, d8):
        pltpu.make_async_copy(frows[s], acc.at[dstc.at[d8]], ssem[s]).wait()

    for i in range(3):
        unpack_idx(i // 2, (i % 2) * SUB, i, i)
        fire_gather(i)
    for jj in range(8):
        i, s = jj % 4, jj % 2
        wait_gather(i)
        nxt = jj + 3
        unpack_idx(nxt // 2, (nxt % 2) * SUB, nxt % 4, nxt % 8)
        fire_gather(nxt % 4)
        if jj >= 2:
            wait_scatter(s, (jj - 2) % 8)
        convert(i, s)
        fire_scatter(s, jj % 8)

    def group_body(k, _):
        for i in range(8):
            gi, s = i % 4, i % 2
            wait_gather(gi)

            @pl.when(8 * k + i + 3 < NSUB)
            def _():
                unpack_idx(4 * k + (i + 3) // 2, ((i + 3) % 2) * SUB,
                           (i + 3) % 4, (i + 3) % 8)
                fire_gather((i + 3) % 4)

            wait_scatter(s, (i - 2) % 8)
            convert(gi, s)
            fire_scatter(s, i)
        return 0
    lax.fori_loop(1, NSUB // 8, group_body, 0)
    wait_scatter(0, 6)
    wait_scatter(1, 7)
    plsc.subcore_barrier()

    pltpu.sync_copy(acc.at[pl.ds(sub * ZROWS, ZROWS)],
                    out_hbm.at[core, pl.ds(sub * ZROWS, ZROWS)])


def _seg_sum(table_bf, pk):
    mesh = plsc.VectorSubcoreMesh(core_axis_name="c", subcore_axis_name="s")
    kern = pl.kernel(
        _seg_sum_body,
        out_type=jax.ShapeDtypeStruct((NC, NPAD, DIM), jnp.float32),
        mesh=mesh,
        compiler_params=pltpu.CompilerParams(use_tc_tiling_on_sc=False, needs_layout_passes=False),
        scratch_types=[
            pltpu.VMEM((NCHUNK, CHUNK), jnp.int32),
            pltpu.VMEM((4, SUB), jnp.int32),
            pltpu.VMEM((8, SUB), jnp.int32),
            pltpu.VMEM((SUB, DIM // 2), jnp.int32),
            pltpu.VMEM((SUB, DIM // 2), jnp.int32),
            pltpu.VMEM((SUB, DIM // 2), jnp.int32),
            pltpu.VMEM((SUB, DIM // 2), jnp.int32),
            pltpu.VMEM((SUB, DIM), jnp.float32),
            pltpu.VMEM((SUB, DIM), jnp.float32),
            pltpu.VMEM_SHARED((NPAD, DIM), jnp.float32),
            pltpu.SemaphoreType.DMA,
            pltpu.SemaphoreType.DMA,
            pltpu.SemaphoreType.DMA,
            pltpu.SemaphoreType.DMA,
            pltpu.SemaphoreType.DMA,
            pltpu.SemaphoreType.DMA,
        ],
    )
    return kern(table_bf, pk)


def _bf16i(x):
    t = x.astype(jnp.bfloat16).reshape(N, DIM // 32, 2, 16)
    t = t.transpose(0, 1, 3, 2).reshape(N, DIM // 2, 2)
    return lax.bitcast_convert_type(t, jnp.int32)


def _pad_edges(src, dst):
    pad = EPAD - E
    i = jnp.arange(pad, dtype=jnp.int32)
    src_p = jnp.concatenate([src, i % N])
    dst_p = jnp.concatenate([dst, N + i % (NPAD - N)])
    return (src_p | (dst_p << 14)).reshape(NW, NCHUNK, CHUNK)


def _dense_bn_body(x_ref, a_ref, w_ref, b_ref, g_ref, be_ref, o_ref):
    h = x_ref[...] + a_ref[0, :N, :] + a_ref[1, :N, :]
    y = jnp.dot(h, w_ref[...], preferred_element_type=jnp.float32) + b_ref[...]
    mu = jnp.mean(y, axis=0, keepdims=True)
    var = jnp.mean((y - mu) ** 2, axis=0, keepdims=True)
    yn = g_ref[...] * (y - mu) / jnp.sqrt(var + 1e-5) + be_ref[...]
    o_ref[...] = jnp.maximum(yn, 0.0)


def _dense_bn(x, agg, w, b, g, be):
    h = w.shape[1]
    return pl.pallas_call(
        _dense_bn_body,
        out_shape=jax.ShapeDtypeStruct((N, h), jnp.float32),
    )(x, agg, w, b.reshape(1, h), g.reshape(1, h), be.reshape(1, h))


def _dense_relu_body(x_ref, a_ref, w_ref, b_ref, o_ref):
    h = x_ref[...] + a_ref[0, :N, :64] + a_ref[1, :N, :64]
    y = jnp.dot(h, w_ref[...], preferred_element_type=jnp.float32) + b_ref[...]
    o_ref[...] = jnp.maximum(y, 0.0)


def _dense_relu(x, agg, w, b):
    h = w.shape[1]
    return pl.pallas_call(
        _dense_relu_body,
        out_shape=jax.ShapeDtypeStruct((N, h), jnp.float32),
    )(x, agg, w, b.reshape(1, h))


def kernel(ft, et, fs, es, W_at, b_at, g_at, be_at, W_as, b_as, g_as, be_as,
           W_ex, b_ex, g_ex, be_ex, W_rt, b_rt):
    ft0 = ft[0]
    pk_t = _pad_edges(et[0, 0], et[0, 1])
    pk_s = _pad_edges(es[0], es[1])

    agg = _seg_sum(_bf16i(ft0), pk_t)
    aligned_t = _dense_bn(ft0, agg, W_at, b_at, g_at, be_at)

    agg = _seg_sum(_bf16i(fs), pk_s)
    aligned_s = _dense_bn(fs, agg, W_as, b_as, g_as, be_as)

    W_ex_p = jnp.pad(W_ex, ((0, 0), (0, 64)))
    b_ex_p = jnp.pad(b_ex, (0, 64))
    g_ex_p = jnp.pad(g_ex, (0, 64))
    be_ex_p = jnp.pad(be_ex, (0, 64))

    agg = _seg_sum(_bf16i(aligned_t), pk_t)
    ht0_pad = _dense_bn(aligned_t, agg, W_ex_p, b_ex_p, g_ex_p, be_ex_p)
    ht0 = ht0_pad[:, :64]

    agg = _seg_sum(_bf16i(aligned_s), pk_s)
    hs = _dense_bn(aligned_s, agg, W_ex, b_ex, g_ex, be_ex)

    agg = _seg_sum(_bf16i(ht0_pad), pk_t)
    ft_rec0 = _dense_relu(ht0, agg, W_rt, b_rt)

    return (hs, ht0, ft_rec0, ft0)


# speedup vs baseline: 2.1131x; 2.1131x over previous
import functools

import jax
import jax.numpy as jnp
from jax import lax
from jax.experimental import pallas as pl
from jax.experimental.pallas import tpu as pltpu
from jax.experimental.pallas import tpu_sc as plsc

N = 10000
E = 320000
NC = 2
NS = 16
NW = NC * NS
CHUNK = 128
NCHUNK = 80
SUBCHUNK = 64
NSUB = 160
EPW = NCHUNK * CHUNK
EPAD = NW * EPW
NPAD = 10240
ZROWS = NPAD // NS


def _seg_sum_body(table_hbm, pk_hbm, out_hbm,
                  pk_idx, srcc, dstc, rows0, rows1, rows2, rows3, acc,
                  sem0, sem1, sem2, sem3):
    core = lax.axis_index("c")
    sub = lax.axis_index("s")
    wid = core * NS + sub
    dim = rows0.shape[1]
    bufs = (rows0, rows1, rows2, rows3)
    sems = (sem0, sem1, sem2, sem3)

    pltpu.sync_copy(pk_hbm.at[wid], pk_idx)

    def zrow(r, _):
        for cc in range(dim // 16):
            rows1[r, pl.ds(cc * 16, 16)] = jnp.zeros((16,), jnp.float32)
        return 0
    lax.fori_loop(0, SUBCHUNK, zrow, 0)

    def unpack(jj, slot):
        j2 = jj // 2
        off = (jj % 2) * SUBCHUNK
        for v in range(SUBCHUNK // 16):
            p = pk_idx[j2, pl.ds(off + v * 16, 16)]
            srcc[slot, pl.ds(v * 16, 16)] = p & 0x3FFF
            dstc[slot, pl.ds(v * 16, 16)] = p >> 14

    def fire(slot):
        pltpu.async_copy(table_hbm.at[srcc.at[slot]], bufs[slot], sems[slot])

    def wait(slot):
        pltpu.make_async_copy(table_hbm.at[srcc.at[slot]], bufs[slot],
                              sems[slot]).wait()

    unpack(0, 0)
    fire(0)
    for k in range(ZROWS // SUBCHUNK):
        pltpu.sync_copy(rows1, acc.at[pl.ds(sub * ZROWS + k * SUBCHUNK, SUBCHUNK)])
    plsc.subcore_barrier()
    for i in range(1, 3):
        unpack(i, i)
        fire(i)

    def quad_body(k, _):
        j = 4 * k
        for i in range(4):
            jj = j + i
            wait(i)

            @pl.when(jj + 3 < NSUB)
            def _():
                unpack(jj + 3, (i + 3) % 4)
                fire((i + 3) % 4)

            pltpu.sync_copy(bufs[i], acc.at[dstc.at[i]], add=True)
        return 0
    lax.fori_loop(0, NSUB // 4, quad_body, 0)
    plsc.subcore_barrier()

    pltpu.sync_copy(acc.at[pl.ds(sub * ZROWS, ZROWS)],
                    out_hbm.at[core, pl.ds(sub * ZROWS, ZROWS)])


def _seg_sum(table, pk, dim):
    mesh = plsc.VectorSubcoreMesh(core_axis_name="c", subcore_axis_name="s")
    kern = pl.kernel(
        _seg_sum_body,
        out_type=jax.ShapeDtypeStruct((NC, NPAD, dim), jnp.float32),
        mesh=mesh,
        scratch_types=[
            pltpu.VMEM((NCHUNK, CHUNK), jnp.int32),
            pltpu.VMEM((4, SUBCHUNK), jnp.int32),
            pltpu.VMEM((4, SUBCHUNK), jnp.int32),
            pltpu.VMEM((SUBCHUNK, dim), jnp.float32),
            pltpu.VMEM((SUBCHUNK, dim), jnp.float32),
            pltpu.VMEM((SUBCHUNK, dim), jnp.float32),
            pltpu.VMEM((SUBCHUNK, dim), jnp.float32),
            pltpu.VMEM_SHARED((NPAD, dim), jnp.float32),
            pltpu.SemaphoreType.DMA,
            pltpu.SemaphoreType.DMA,
            pltpu.SemaphoreType.DMA,
            pltpu.SemaphoreType.DMA,
        ],
    )
    return kern(table, pk)


def _pad_edges(src, dst):
    pad = EPAD - E
    i = jnp.arange(pad, dtype=jnp.int32)
    src_p = jnp.concatenate([src, i % N])
    dst_p = jnp.concatenate([dst, N + i % (NPAD - N)])
    return (src_p | (dst_p << 14)).reshape(NW, NCHUNK, CHUNK)


def _dense_bn_body(x_ref, a_ref, w_ref, b_ref, g_ref, be_ref, o_ref):
    h = x_ref[...] + a_ref[0, :N, :] + a_ref[1, :N, :]
    y = jnp.dot(h, w_ref[...], preferred_element_type=jnp.float32) + b_ref[...]
    mu = jnp.mean(y, axis=0, keepdims=True)
    var = jnp.mean((y - mu) ** 2, axis=0, keepdims=True)
    yn = g_ref[...] * (y - mu) / jnp.sqrt(var + 1e-5) + be_ref[...]
    o_ref[...] = jnp.maximum(yn, 0.0)


def _dense_bn(x, agg, w, b, g, be):
    h = w.shape[1]
    return pl.pallas_call(
        _dense_bn_body,
        out_shape=jax.ShapeDtypeStruct((N, h), jnp.float32),
    )(x, agg, w, b.reshape(1, h), g.reshape(1, h), be.reshape(1, h))


def _dense_relu_body(x_ref, a_ref, w_ref, b_ref, o_ref):
    h = x_ref[...] + a_ref[0, :N, :64] + a_ref[1, :N, :64]
    y = jnp.dot(h, w_ref[...], preferred_element_type=jnp.float32) + b_ref[...]
    o_ref[...] = jnp.maximum(y, 0.0)


def _dense_relu(x, agg, w, b):
    h = w.shape[1]
    return pl.pallas_call(
        _dense_relu_body,
        out_shape=jax.ShapeDtypeStruct((N, h), jnp.float32),
    )(x, agg, w, b.reshape(1, h))


def kernel(ft, et, fs, es, W_at, b_at, g_at, be_at, W_as, b_as, g_as, be_as,
           W_ex, b_ex, g_ex, be_ex, W_rt, b_rt):
    ft0 = ft[0]
    pk_t = _pad_edges(et[0, 0], et[0, 1])
    pk_s = _pad_edges(es[0], es[1])

    agg = _seg_sum(ft0, pk_t, dim=128)
    aligned_t = _dense_bn(ft0, agg, W_at, b_at, g_at, be_at)

    agg = _seg_sum(fs, pk_s, dim=128)
    aligned_s = _dense_bn(fs, agg, W_as, b_as, g_as, be_as)

    W_ex_p = jnp.pad(W_ex, ((0, 0), (0, 64)))
    b_ex_p = jnp.pad(b_ex, (0, 64))
    g_ex_p = jnp.pad(g_ex, (0, 64))
    be_ex_p = jnp.pad(be_ex, (0, 64))

    agg = _seg_sum(aligned_t, pk_t, dim=128)
    ht0_pad = _dense_bn(aligned_t, agg, W_ex_p, b_ex_p, g_ex_p, be_ex_p)
    ht0 = ht0_pad[:, :64]

    agg = _seg_sum(aligned_s, pk_s, dim=128)
    hs = _dense_bn(aligned_s, agg, W_ex, b_ex, g_ex, be_ex)

    agg = _seg_sum(ht0_pad, pk_t, dim=128)
    ft_rec0 = _dense_relu(ht0, agg, W_rt, b_rt)

    return (hs, ht0, ft_rec0, ft0)
